# baseline (speedup 1.0000x reference)
import functools

import jax
import jax.numpy as jnp
from jax import lax
from jax.experimental import pallas as pl
from jax.experimental.pallas import tpu as pltpu
from jax.experimental.pallas import tpu_sc as plsc

NC, NS, L = 2, 16, 16
NW = NC * NS
BATCH, D, V = 16384, 512, 1000
PW = D // 32
B_PER_W = BATCH // NW
RPT = 64
BR = 512

_mesh = plsc.VectorSubcoreMesh(core_axis_name="c", subcore_axis_name="s")
_params = pltpu.CompilerParams(needs_layout_passes=False)

_BIT = [1 << c for c in range(32)]


@functools.partial(
    pl.kernel,
    out_type=jax.ShapeDtypeStruct((BATCH * PW,), jnp.int32),
    mesh=_mesh,
    compiler_params=_params,
    scratch_types=[
        pltpu.HBM((NC * V * PW,), jnp.int32),
        pltpu.VMEM((RPT, D), jnp.float32),
        pltpu.VMEM((RPT * PW,), jnp.int32),
        pltpu.VMEM((V * PW,), jnp.int32),
        pltpu.VMEM((B_PER_W,), jnp.int32),
        pltpu.VMEM((B_PER_W * PW,), jnp.int32),
        pltpu.SemaphoreType.DMA,
    ],
)
def _sc_gather(lbl_hbm, w_hbm, pcode_hbm,
               packed_hbm, w_v, pk_v, ptab_v, lbl_v, pc_v, sl):
    cid = lax.axis_index("c")
    sid = lax.axis_index("s")
    base_w = (sid * NC + cid) * B_PER_W
    lane_ids = lax.iota(jnp.int32, L)

    lbl_cp = pltpu.make_async_copy(
        lbl_hbm.at[pl.ds(base_w, B_PER_W)], lbl_v, sl)
    lbl_cp.start()

    start = jnp.minimum(sid * RPT, V - RPT)
    pltpu.sync_copy(w_hbm.at[pl.ds(start, RPT)], w_v)

    def pack_row(r, _):
        bits = jnp.zeros((L,), jnp.uint32)
        for c in range(32):
            wv = w_v[r, pl.ds(c * L, L)]
            bits = bits | jnp.where(wv > 0.5, jnp.uint32(_BIT[c]),
                                    jnp.uint32(0))
        pk_v[pl.ds(r * PW, PW)] = plsc.bitcast(bits, jnp.int32)
        return 0

    lax.fori_loop(0, RPT, pack_row, 0)
    pltpu.sync_copy(pk_v,
                    packed_hbm.at[pl.ds(cid * V * PW + start * PW, RPT * PW)])
    plsc.subcore_barrier()
    pltpu.sync_copy(packed_hbm.at[pl.ds(cid * V * PW, V * PW)], ptab_v)
    lbl_cp.wait()

    def row_body(i, _):
        row_splat = plsc.load_gather(
            lbl_v, [jnp.full((L,), i, jnp.int32)])
        pc_v[pl.ds(i * PW, PW)] = plsc.load_gather(
            ptab_v, [row_splat * PW + lane_ids])
        return 0

    lax.fori_loop(0, B_PER_W, row_body, 0)
    pltpu.sync_copy(pc_v, pcode_hbm.at[pl.ds(base_w * PW, B_PER_W * PW)])


def _tc_body(x_ref, pc_ref, o_ref):
    pc = lax.bitcast_convert_type(pc_ref[...], jnp.uint32)
    pct = jnp.concatenate([pc] * (D // PW), axis=1)
    sh = lax.broadcasted_iota(jnp.uint32, (BR, D), 1) // jnp.uint32(L)
    m = (pct >> sh) & jnp.uint32(1)
    o_ref[...] = x_ref[...] * m.astype(jnp.float32)


_tc_mul = pl.pallas_call(
    _tc_body,
    grid=(BATCH // BR,),
    in_specs=[
        pl.BlockSpec((BR, D), lambda i: (i, 0)),
        pl.BlockSpec((BR, PW), lambda i: (i, 0)),
    ],
    out_specs=pl.BlockSpec((BR, D), lambda i: (i, 0)),
    out_shape=jax.ShapeDtypeStruct((BATCH, D), jnp.float32),
)


def kernel(x, label, weight):
    pcode = _sc_gather(label, weight).reshape(BATCH, PW)
    return _tc_mul(x, pcode)


# speedup vs baseline: 1.2749x; 1.2749x over previous
import functools

import jax
import jax.numpy as jnp
from jax import lax
from jax.experimental import pallas as pl
from jax.experimental.pallas import tpu as pltpu
from jax.experimental.pallas import tpu_sc as plsc

NC, NS, L = 2, 16, 16
NW = NC * NS
BATCH, D, V = 16384, 512, 1000
PW = D // 32
NCHUNK_F32 = D // L
B_PER_W = BATCH // NW
CHUNK = 32
NBUF = 3
NG = B_PER_W // CHUNK
RPT = 64

_mesh = plsc.VectorSubcoreMesh(core_axis_name="c", subcore_axis_name="s")
_params = pltpu.CompilerParams(needs_layout_passes=False)

_BIT = [1 << c for c in range(32)]


@functools.partial(
    pl.kernel,
    out_type=jax.ShapeDtypeStruct((BATCH, D), jnp.float32),
    mesh=_mesh,
    compiler_params=_params,
    scratch_types=[
        pltpu.HBM((NC * V * PW,), jnp.int32),
        pltpu.VMEM((RPT * PW,), jnp.int32),
        pltpu.VMEM((V * PW,), jnp.int32),
        pltpu.VMEM((B_PER_W,), jnp.int32),
        pltpu.VMEM((NBUF, CHUNK, D), jnp.float32),
        pltpu.VMEM((NBUF, CHUNK, D), jnp.float32),
        pltpu.SemaphoreType.DMA,
        pltpu.SemaphoreType.DMA,
        pltpu.SemaphoreType.DMA,
        pltpu.SemaphoreType.DMA,
        pltpu.SemaphoreType.DMA,
        pltpu.SemaphoreType.DMA,
        pltpu.SemaphoreType.DMA,
    ],
)
def _sc_kernel(x_hbm, lbl_hbm, w_hbm, out_hbm,
               packed_hbm, pk_v, ptab_v, lbl_v, x_v, out_v,
               sx0, sx1, sx2, so0, so1, so2, sl):
    cid = lax.axis_index("c")
    sid = lax.axis_index("s")
    base_w = (sid * NC + cid) * B_PER_W
    lane_ids = lax.iota(jnp.int32, L)
    sx, so = (sx0, sx1, sx2), (so0, so1, so2)

    def x_copy(g):
        return pltpu.make_async_copy(
            x_hbm.at[pl.ds(base_w + g * CHUNK, CHUNK)], x_v.at[g % NBUF],
            sx[g % NBUF])

    def out_copy(g):
        return pltpu.make_async_copy(
            out_v.at[g % NBUF], out_hbm.at[pl.ds(base_w + g * CHUNK, CHUNK)],
            so[g % NBUF])

    lbl_cp = pltpu.make_async_copy(
        lbl_hbm.at[pl.ds(base_w, B_PER_W)], lbl_v, sl)
    lbl_cp.start()
    for g in range(NBUF):
        x_copy(g).start()

    start = jnp.minimum(sid * RPT, V - RPT)
    pltpu.sync_copy(w_hbm.at[pl.ds(start, CHUNK)], out_v.at[0])
    pltpu.sync_copy(w_hbm.at[pl.ds(start + CHUNK, CHUNK)], out_v.at[1])

    def pack_row(r, _):
        bits = jnp.zeros((L,), jnp.uint32)
        for c in range(32):
            wv = out_v[r >> 5, r & 31, pl.ds(c * L, L)]
            bits = bits | jnp.where(wv > 0.5, jnp.uint32(_BIT[c]),
                                    jnp.uint32(0))
        pk_v[pl.ds(r * PW, PW)] = plsc.bitcast(bits, jnp.int32)
        return 0

    lax.fori_loop(0, RPT, pack_row, 0)
    pltpu.sync_copy(pk_v,
                    packed_hbm.at[pl.ds(cid * V * PW + start * PW, RPT * PW)])
    plsc.subcore_barrier()
    pltpu.sync_copy(packed_hbm.at[pl.ds(cid * V * PW, V * PW)], ptab_v)
    lbl_cp.wait()

    def make_pv(idx):
        row_splat = plsc.load_gather(
            lbl_v, [jnp.full((L,), idx, jnp.int32)])
        return plsc.bitcast(
            plsc.load_gather(ptab_v, [row_splat * PW + lane_ids]),
            jnp.uint32)

    for g in range(NG):
        b = g % NBUF
        x_copy(g).wait()
        if g >= NBUF:
            out_copy(g - NBUF).wait()

        def row_body(i, _):
            i2 = i * 2
            pv0 = make_pv(g * CHUNK + i2)
            pv1 = make_pv(g * CHUNK + i2 + 1)
            for c in range(NCHUNK_F32):
                xv0 = x_v[b, i2, pl.ds(c * L, L)]
                m0 = (pv0 & jnp.uint32(_BIT[c])) != 0
                out_v[b, i2, pl.ds(c * L, L)] = jnp.where(m0, xv0, 0.0)
                xv1 = x_v[b, i2 + 1, pl.ds(c * L, L)]
                m1 = (pv1 & jnp.uint32(_BIT[c])) != 0
                out_v[b, i2 + 1, pl.ds(c * L, L)] = jnp.where(m1, xv1, 0.0)
            return 0

        lax.fori_loop(0, CHUNK // 2, row_body, 0)
        out_copy(g).start()
        if g + NBUF < NG:
            x_copy(g + NBUF).start()
    for g in range(NG - NBUF, NG):
        out_copy(g).wait()


def kernel(x, label, weight):
    return _sc_kernel(x, label, weight)


# speedup vs baseline: 1.4385x; 1.1283x over previous
import functools

import jax
import jax.numpy as jnp
from jax import lax
from jax.experimental import pallas as pl
from jax.experimental.pallas import tpu as pltpu
from jax.experimental.pallas import tpu_sc as plsc

NC, NS, L = 2, 16, 16
NW = NC * NS
BATCH, D, V = 16384, 512, 1000
PW = D // 32
NCHUNK_F32 = D // L
B_PER_W = BATCH // NW
CHUNK = 32
NBUF = 3
NG = B_PER_W // CHUNK
RPT = 64

_mesh = plsc.VectorSubcoreMesh(core_axis_name="c", subcore_axis_name="s")
_params = pltpu.CompilerParams(needs_layout_passes=False)

_BIT = [1 << c for c in range(32)]


@functools.partial(
    pl.kernel,
    out_type=jax.ShapeDtypeStruct((BATCH, D), jnp.float32),
    mesh=_mesh,
    compiler_params=_params,
    scratch_types=[
        pltpu.HBM((NC * V * PW,), jnp.int32),
        pltpu.VMEM((RPT * PW,), jnp.int32),
        pltpu.VMEM((V * PW,), jnp.int32),
        pltpu.VMEM((B_PER_W,), jnp.int32),
        pltpu.VMEM((NBUF, CHUNK, D), jnp.float32),
        pltpu.VMEM((NBUF, CHUNK, D), jnp.float32),
        pltpu.SemaphoreType.DMA,
        pltpu.SemaphoreType.DMA,
        pltpu.SemaphoreType.DMA,
        pltpu.SemaphoreType.DMA,
        pltpu.SemaphoreType.DMA,
        pltpu.SemaphoreType.DMA,
        pltpu.SemaphoreType.DMA,
    ],
)
def _sc_kernel(x_hbm, lbl_hbm, w_hbm, out_hbm,
               packed_hbm, pk_v, ptab_v, lbl_v, x_v, out_v,
               sx0, sx1, sx2, so0, so1, so2, sl):
    cid = lax.axis_index("c")
    sid = lax.axis_index("s")
    base_w = (sid * NC + cid) * B_PER_W
    lane_ids = lax.iota(jnp.int32, L)
    sx, so = (sx0, sx1, sx2), (so0, so1, so2)

    def x_copy(g):
        return pltpu.make_async_copy(
            x_hbm.at[pl.ds(base_w + g * CHUNK, CHUNK)], x_v.at[g % NBUF],
            sx[g % NBUF])

    def out_copy(g):
        return pltpu.make_async_copy(
            out_v.at[g % NBUF], out_hbm.at[pl.ds(base_w + g * CHUNK, CHUNK)],
            so[g % NBUF])

    lbl_cp = pltpu.make_async_copy(
        lbl_hbm.at[pl.ds(base_w, B_PER_W)], lbl_v, sl)
    lbl_cp.start()
    for g in range(NBUF):
        x_copy(g).start()

    start = jnp.minimum(sid * RPT, V - RPT)
    pltpu.sync_copy(w_hbm.at[pl.ds(start, CHUNK)], out_v.at[0])
    pltpu.sync_copy(w_hbm.at[pl.ds(start + CHUNK, CHUNK)], out_v.at[1])

    def pack_row(r, _):
        bits = jnp.zeros((L,), jnp.uint32)
        for c in range(32):
            wv = out_v[r >> 5, r & 31, pl.ds(c * L, L)]
            bits = bits | jnp.where(wv > 0.5, jnp.uint32(_BIT[c]),
                                    jnp.uint32(0))
        pk_v[pl.ds(r * PW, PW)] = plsc.bitcast(bits, jnp.int32)
        return 0

    lax.fori_loop(0, RPT, pack_row, 0)
    pltpu.sync_copy(pk_v,
                    packed_hbm.at[pl.ds(cid * V * PW + start * PW, RPT * PW)])
    plsc.subcore_barrier()
    pltpu.sync_copy(packed_hbm.at[pl.ds(cid * V * PW, V * PW)], ptab_v)
    lbl_cp.wait()

    def make_pv(idx):
        row_splat = plsc.load_gather(
            lbl_v, [jnp.full((L,), idx, jnp.int32)])
        return plsc.bitcast(
            plsc.load_gather(ptab_v, [row_splat * PW + lane_ids]),
            jnp.uint32)

    for g in range(NG):
        b = g % NBUF
        x_copy(g).wait()
        if g >= NBUF:
            out_copy(g - NBUF).wait()

        def row_body(i, _):
            pv = make_pv(g * CHUNK + i)
            for c in range(NCHUNK_F32):
                xv = x_v[b, i, pl.ds(c * L, L)]
                m = (pv & jnp.uint32(_BIT[c])) != 0
                out_v[b, i, pl.ds(c * L, L)] = jnp.where(m, xv, 0.0)
            return 0

        lax.fori_loop(0, CHUNK, row_body, 0)
        out_copy(g).start()
        if g + NBUF < NG:
            x_copy(g + NBUF).start()
    for g in range(NG - NBUF, NG):
        out_copy(g).wait()


def kernel(x, label, weight):
    return _sc_kernel(x, label, weight)
